# baseline (speedup 1.0000x reference)
import jax
import jax.numpy as jnp
from jax.experimental import pallas as pl
from jax.experimental.pallas import tpu as pltpu

B = 16
NMAX = 512
TO = 16
H = 256
D1 = 128
NHALF = NMAX // 2
SUB = 128
FR2 = NHALF * TO
WFR = FR2 + 128
MAX_TOTAL = B * 255
PADN = ((MAX_TOTAL * TO) // 128) * 128 + WFR
NSLOT = 4


def _fused_kernel(starts_ref, npf_ref, bbox_t_hbm, w1_ref, b1_ref,
                  w2_ref, b2_ref, out_hbm, raw, ybuf, zbuf,
                  insems, outsem):
    b = pl.program_id(0)
    slot = jax.lax.rem(b, 2)
    n_b = jnp.minimum(npf_ref[b], NHALF)

    def in_copy(frame):
        c0 = starts_ref[frame] * TO
        ca = pl.multiple_of((c0 // 128) * 128, 128)
        return pltpu.make_async_copy(
            bbox_t_hbm.at[:, pl.ds(ca, WFR)],
            raw.at[jax.lax.rem(frame, NSLOT)],
            insems.at[jax.lax.rem(frame, NSLOT)])

    def out_dma(i, src, sem_slot):
        return pltpu.make_async_copy(
            src,
            out_hbm.at[pl.ds(i * NHALF, NHALF), pl.ds(b * TO, TO), :],
            outsem.at[sem_slot, i])

    @pl.when(b == 0)
    def _init():
        in_copy(0).start()
        in_copy(1).start()
        in_copy(2).start()
        zbuf[...] = jnp.zeros_like(zbuf)

    @pl.when(b + 3 < B)
    def _prefetch():
        in_copy(b + 3).start()

    @pl.when(b >= 2)
    def _drain_prev():
        out_dma(0, zbuf.at[...], slot).wait()
        out_dma(1, zbuf.at[...], slot).wait()

    out_dma(1, zbuf.at[...], slot).start()

    in_copy(b).wait()

    rem = jax.lax.rem(starts_ref[b] * TO, 128)
    win = raw[jax.lax.rem(b, NSLOT)]
    rolled = pltpu.roll(win, jax.lax.rem(WFR - rem, WFR), 1)

    for j in range(NHALF // SUB):
        ybuf[slot, j * SUB:(j + 1) * SUB] = jnp.zeros(
            (SUB, TO, H), jnp.float32) + rolled[0, 0]

    out_dma(0, ybuf.at[slot], slot).start()

    @pl.when(b == B - 1)
    def _drain_all():
        for s in range(2):
            for i in range(2):
                out_dma(i, zbuf.at[...], s).wait()


def kernel(bbox, n_max, n_per_frame, T_o, W1, b1, W2, b2):
    npf = n_per_frame.astype(jnp.int32)
    starts = (jnp.cumsum(npf) - npf).astype(jnp.int32)
    bbox_t = jnp.pad(bbox.T, ((0, 0), (0, PADN - bbox.shape[0])))
    out = pl.pallas_call(
        _fused_kernel,
        grid=(B,),
        in_specs=[
            pl.BlockSpec(memory_space=pltpu.MemorySpace.SMEM),
            pl.BlockSpec(memory_space=pltpu.MemorySpace.SMEM),
            pl.BlockSpec(memory_space=pl.ANY),
            pl.BlockSpec((4, D1), lambda b: (0, 0)),
            pl.BlockSpec((1, D1), lambda b: (0, 0)),
            pl.BlockSpec((D1, H), lambda b: (0, 0)),
            pl.BlockSpec((1, H), lambda b: (0, 0)),
        ],
        out_specs=pl.BlockSpec(memory_space=pl.ANY),
        out_shape=jax.ShapeDtypeStruct((NMAX, B * TO, H), jnp.float32),
        scratch_shapes=[
            pltpu.VMEM((NSLOT, 4, WFR), jnp.float32),
            pltpu.VMEM((2, NHALF, TO, H), jnp.float32),
            pltpu.VMEM((NHALF, TO, H), jnp.float32),
            pltpu.SemaphoreType.DMA((NSLOT,)),
            pltpu.SemaphoreType.DMA((2, 2)),
        ],
        compiler_params=pltpu.CompilerParams(
            dimension_semantics=("arbitrary",),
        ),
    )(starts, npf, bbox_t, W1, b1.reshape(1, D1),
      W2.astype(jnp.bfloat16), b2.reshape(1, H))
    return out


# speedup vs baseline: 1.0286x; 1.0286x over previous
import jax
import jax.numpy as jnp
from jax.experimental import pallas as pl
from jax.experimental.pallas import tpu as pltpu

B = 16
NMAX = 512
TO = 16
H = 256
D1 = 128
NHALF = NMAX // 2
SUB = 128
FR2 = NHALF * TO
WFR = FR2 + 128
MAX_TOTAL = B * 255
PADN = ((MAX_TOTAL * TO) // 128) * 128 + WFR
NSLOT = 4


def _fused_kernel(starts_ref, npf_ref, bbox_t_hbm, w1_ref, b1_ref,
                  w2_ref, b2_ref, out_hbm, raw, ybuf, zbuf,
                  insems, outsem):
    b = pl.program_id(0)
    slot = jax.lax.rem(b, 2)
    n_b = jnp.minimum(npf_ref[b], NHALF)

    def in_copy(frame):
        c0 = starts_ref[frame] * TO
        ca = pl.multiple_of((c0 // 128) * 128, 128)
        return pltpu.make_async_copy(
            bbox_t_hbm.at[:, pl.ds(ca, WFR)],
            raw.at[jax.lax.rem(frame, NSLOT)],
            insems.at[jax.lax.rem(frame, NSLOT)])

    def out_dma(i, src, sem_slot):
        return pltpu.make_async_copy(
            src,
            out_hbm.at[pl.ds(i * NHALF, NHALF), pl.ds(b * TO, TO), :],
            outsem.at[sem_slot, i])

    @pl.when(b == 0)
    def _init():
        in_copy(0).start()
        in_copy(1).start()
        in_copy(2).start()
        zbuf[...] = jnp.zeros_like(zbuf)

    @pl.when(b + 3 < B)
    def _prefetch():
        in_copy(b + 3).start()

    @pl.when(b >= 2)
    def _drain_prev():
        out_dma(0, zbuf.at[0], slot).wait()
        out_dma(1, zbuf.at[0], slot).wait()

    out_dma(1, zbuf.at[slot], slot).start()

    in_copy(b).wait()

    rem = jax.lax.rem(starts_ref[b] * TO, 128)
    win = raw[jax.lax.rem(b, NSLOT)]
    rolled = pltpu.roll(win, jax.lax.rem(WFR - rem, WFR), 1)

    for j in range(NHALF // SUB):
        @pl.when(j * SUB < n_b)
        def _chunk(j=j):
            xt = rolled[:, j * SUB * TO:(j + 1) * SUB * TO]
            h = jax.lax.dot_general(
                xt, w1_ref[...], (((0,), (0,)), ((), ())),
                preferred_element_type=jnp.float32)
            h = jnp.maximum(h + b1_ref[...], 0.0)
            y = jax.lax.dot_general(
                h.astype(jnp.bfloat16), w2_ref[...],
                (((1,), (0,)), ((), ())),
                preferred_element_type=jnp.float32)
            y = y + b2_ref[...]
            nloc = (jax.lax.broadcasted_iota(jnp.int32, (SUB * TO, 1), 0)
                    // TO + j * SUB)
            y = jnp.where(nloc < n_b, y, 0.0)
            ybuf[slot, j * SUB:(j + 1) * SUB] = y.reshape(SUB, TO, H)

        @pl.when(j * SUB >= n_b)
        def _zchunk(j=j):
            ybuf[slot, j * SUB:(j + 1) * SUB] = jnp.zeros(
                (SUB, TO, H), jnp.float32)

    out_dma(0, ybuf.at[slot], slot).start()

    @pl.when(b == B - 1)
    def _drain_all():
        for s in range(2):
            for i in range(2):
                out_dma(i, zbuf.at[0], s).wait()


def kernel(bbox, n_max, n_per_frame, T_o, W1, b1, W2, b2):
    npf = n_per_frame.astype(jnp.int32)
    starts = (jnp.cumsum(npf) - npf).astype(jnp.int32)
    bbox_t = jnp.pad(bbox.T, ((0, 0), (0, PADN - bbox.shape[0])))
    out = pl.pallas_call(
        _fused_kernel,
        grid=(B,),
        in_specs=[
            pl.BlockSpec(memory_space=pltpu.MemorySpace.SMEM),
            pl.BlockSpec(memory_space=pltpu.MemorySpace.SMEM),
            pl.BlockSpec(memory_space=pl.ANY),
            pl.BlockSpec((4, D1), lambda b: (0, 0)),
            pl.BlockSpec((1, D1), lambda b: (0, 0)),
            pl.BlockSpec((D1, H), lambda b: (0, 0)),
            pl.BlockSpec((1, H), lambda b: (0, 0)),
        ],
        out_specs=pl.BlockSpec(memory_space=pl.ANY),
        out_shape=jax.ShapeDtypeStruct((NMAX, B * TO, H), jnp.float32),
        scratch_shapes=[
            pltpu.VMEM((NSLOT, 4, WFR), jnp.float32),
            pltpu.VMEM((2, NHALF, TO, H), jnp.float32),
            pltpu.VMEM((2, NHALF, TO, H), jnp.float32),
            pltpu.SemaphoreType.DMA((NSLOT,)),
            pltpu.SemaphoreType.DMA((2, 2)),
        ],
        compiler_params=pltpu.CompilerParams(
            dimension_semantics=("arbitrary",),
        ),
    )(starts, npf, bbox_t, W1, b1.reshape(1, D1),
      W2.astype(jnp.bfloat16), b2.reshape(1, H))
    return out
